# baseline (speedup 1.0000x reference)
import functools

import jax
import jax.numpy as jnp
from jax import lax
from jax.experimental import pallas as pl
from jax.experimental.pallas import tpu as pltpu
from jax.experimental.pallas import tpu_sc as plsc


def _sc_gather(users, items, user_table, item_table):
    info = plsc.get_sparse_core_info()
    nw = info.num_cores * info.num_subcores
    batch = users.shape[0]
    embed = user_table.shape[1]
    b_per_w = batch // nw

    mesh = plsc.VectorSubcoreMesh(core_axis_name="c", subcore_axis_name="s")
    row_struct = jax.ShapeDtypeStruct((batch, embed), user_table.dtype)

    @functools.partial(
        pl.kernel,
        mesh=mesh,
        compiler_params=pltpu.CompilerParams(use_tc_tiling_on_sc=False),
        out_type=[row_struct, row_struct],
        scratch_types=[
            pltpu.VMEM((b_per_w,), jnp.int32),
            pltpu.VMEM((b_per_w,), jnp.int32),
            pltpu.VMEM((b_per_w, embed), jnp.float32),
            pltpu.VMEM((b_per_w, embed), jnp.float32),
            pltpu.SemaphoreType.DMA,
            pltpu.SemaphoreType.DMA,
        ],
    )
    def gather_kernel(users_hbm, items_hbm, utab_hbm, itab_hbm,
                      outu_hbm, outi_hbm,
                      idx_u, idx_i, rows_u, rows_i, sem_u, sem_i):
        wid = lax.axis_index("s") * info.num_cores + lax.axis_index("c")
        base = wid * b_per_w
        sl = pl.ds(base, b_per_w)
        pltpu.sync_copy(users_hbm.at[sl], idx_u)
        pltpu.sync_copy(items_hbm.at[sl], idx_i)
        cu = pltpu.async_copy(utab_hbm.at[idx_u], rows_u, sem_u)
        ci = pltpu.async_copy(itab_hbm.at[idx_i], rows_i, sem_i)
        cu.wait()
        pltpu.sync_copy(rows_u, outu_hbm.at[sl])
        ci.wait()
        pltpu.sync_copy(rows_i, outi_hbm.at[sl])

    return gather_kernel(users, items, user_table, item_table)


def _mlp_body(xu_ref, xi_ref, w1u_ref, w1i_ref, b1_ref, w2_ref, b2_ref,
              out_ref):
    dn = (((1,), (1,)), ((), ()))
    h = lax.dot_general(xu_ref[...], w1u_ref[...], dn,
                        preferred_element_type=jnp.float32,
                        precision=lax.Precision.HIGHEST)
    h += lax.dot_general(xi_ref[...], w1i_ref[...], dn,
                         preferred_element_type=jnp.float32,
                         precision=lax.Precision.HIGHEST)
    h = jnp.maximum(h + b1_ref[...], 0.0)
    out = jnp.sum(h * w2_ref[...], axis=1, keepdims=True)
    out_ref[...] = out + b2_ref[0, 0]


def _tc_mlp(xu, xi, W1, b1, W2, b2):
    batch, embed = xu.shape
    hidden = W1.shape[0]
    w1u = W1[:, :embed]
    w1i = W1[:, embed:]
    b1r = b1.reshape(1, hidden)
    b2r = b2.reshape(1, 1)
    blk = 2048
    grid = (batch // blk,)
    return pl.pallas_call(
        _mlp_body,
        grid=grid,
        in_specs=[
            pl.BlockSpec((blk, embed), lambda i: (i, 0)),
            pl.BlockSpec((blk, embed), lambda i: (i, 0)),
            pl.BlockSpec((hidden, embed), lambda i: (0, 0)),
            pl.BlockSpec((hidden, embed), lambda i: (0, 0)),
            pl.BlockSpec((1, hidden), lambda i: (0, 0)),
            pl.BlockSpec((1, hidden), lambda i: (0, 0)),
            pl.BlockSpec((1, 1), lambda i: (0, 0)),
        ],
        out_specs=pl.BlockSpec((blk, 1), lambda i: (i, 0)),
        out_shape=jax.ShapeDtypeStruct((batch, 1), jnp.float32),
    )(xu, xi, w1u, w1i, b1r, W2, b2r)


@jax.jit
def kernel(users, items, user_table, item_table, W1, b1, W2, b2):
    xu, xi = _sc_gather(users, items, user_table, item_table)
    return _tc_mlp(xu, xi, W1, b1, W2, b2)
